# Initial kernel scaffold; baseline (speedup 1.0000x reference)
import jax
import jax.numpy as jnp
from jax.experimental import pallas as pl


def kernel(x, edge_index, edge_type, W, eps, A_table, B_table, W1, b1, W2, b2, W3, b3):
    raise NotImplementedError("write your pallas kernel here")


# speedup vs baseline: 5.9480x; 5.9480x over previous
import functools

import jax
import jax.numpy as jnp
from jax import lax
from jax.experimental import pallas as pl
from jax.experimental.pallas import tpu as pltpu
from jax.experimental.pallas import tpu_sc as plsc

N = 10000
E = 320000
D = 128
RANK = 8
NUM_REL = 64

NC = 2
NS = 16
LANES = 16

CH = 128
NCHUNK = E // CH
NW = NC * NS

NSHARD = 4
SHN = N // NSHARD
CROWS = SHN * NUM_REL
CPAD = 16
ZROWS = CROWS // NS

def _sx_body(x_hbm, src_hbm, dst_hbm, zero_hbm, out_hbm,
             acc, idx_s, idx_d, rows, sem):
    c = lax.axis_index("c")
    s = lax.axis_index("s")
    w = c * NS + s

    ZB = 624
    pltpu.sync_copy(zero_hbm.at[pl.ds(s * ZB, ZB)], acc.at[pl.ds(s * ZB, ZB)])

    @pl.when(s == 0)
    def _():
        pltpu.sync_copy(zero_hbm.at[pl.ds(NS * ZB, N - NS * ZB)],
                        acc.at[pl.ds(NS * ZB, N - NS * ZB)])

    plsc.subcore_barrier()

    nch = (NCHUNK // NW) + (w < (NCHUNK % NW)).astype(jnp.int32)

    @pl.loop(0, nch)
    def _(j):
        base = (j * NW + w) * CH
        pltpu.sync_copy(src_hbm.at[pl.ds(base, CH)], idx_s)
        pltpu.sync_copy(dst_hbm.at[pl.ds(base, CH)], idx_d)
        pltpu.async_copy(x_hbm.at[idx_s], rows, sem).wait()
        pltpu.sync_copy(rows, acc.at[idx_d], add=True)

    plsc.subcore_barrier()
    pltpu.sync_copy(acc.at[pl.ds(s * ZB, ZB)],
                    out_hbm.at[c].at[pl.ds(s * ZB, ZB)])

    @pl.when(s == 0)
    def _():
        pltpu.sync_copy(acc.at[pl.ds(NS * ZB, N - NS * ZB)],
                        out_hbm.at[c].at[pl.ds(NS * ZB, N - NS * ZB)])


def _c_body(p_hbm, src_hbm, dst_hbm, rel_hbm, zero_hbm, out_hbm,
            cacc, idx_s, idx_d, idx_r, gidx, cidx, prow, sem):
    c = lax.axis_index("c")
    s = lax.axis_index("s")

    for p in range(2):
        sh = c * 2 + p
        base_node = sh * SHN

        pltpu.sync_copy(zero_hbm, cacc.at[pl.ds(s * ZROWS, ZROWS)])

        @pl.when(s == 0)
        def _():
            pltpu.sync_copy(zero_hbm.at[pl.ds(0, CPAD)],
                            cacc.at[pl.ds(CROWS, CPAD)])

        plsc.subcore_barrier()

        nch = (NCHUNK // NS) + (s < (NCHUNK % NS)).astype(jnp.int32)

        @pl.loop(0, nch)
        def _(j):
            base = (j * NS + s) * CH
            pltpu.sync_copy(src_hbm.at[pl.ds(base, CH)], idx_s)
            pltpu.sync_copy(dst_hbm.at[pl.ds(base, CH)], idx_d)
            pltpu.sync_copy(rel_hbm.at[pl.ds(base, CH)], idx_r)
            for t in range(CH // LANES):
                sl = pl.ds(t * LANES, LANES)
                sv = idx_s[sl]
                rv = idx_r[sl]
                dv = idx_d[sl]
                gidx[sl] = sv * NUM_REL + rv
                loc = dv - base_node
                ok = (loc >= 0) & (loc < SHN)
                cidx[sl] = jnp.where(ok, loc * NUM_REL + rv, CROWS)
            pltpu.async_copy(p_hbm.at[gidx], prow, sem).wait()
            pltpu.sync_copy(prow, cacc.at[cidx], add=True)

        plsc.subcore_barrier()
        dump_rows = CROWS // NS
        pltpu.sync_copy(cacc.at[pl.ds(s * dump_rows, dump_rows)],
                        out_hbm.at[sh].at[pl.ds(s * dump_rows, dump_rows)])
        plsc.subcore_barrier()


@functools.lru_cache(maxsize=1)
def _sc_kernels():
    vmesh = plsc.VectorSubcoreMesh(core_axis_name="c", subcore_axis_name="s",
                                   num_cores=NC, num_subcores=NS)
    sx_kernel = pl.kernel(
        _sx_body,
        out_type=jax.ShapeDtypeStruct((NC, N, D), jnp.float32),
        mesh=vmesh,
        scratch_types=[
            pltpu.VMEM_SHARED((N, D), jnp.float32),
            pltpu.VMEM((CH,), jnp.int32),
            pltpu.VMEM((CH,), jnp.int32),
            pltpu.VMEM((CH, D), jnp.float32),
            pltpu.SemaphoreType.DMA,
        ],
    )
    c_kernel = pl.kernel(
        _c_body,
        out_type=jax.ShapeDtypeStruct((NSHARD, CROWS, RANK), jnp.float32),
        mesh=vmesh,
        compiler_params=pltpu.CompilerParams(use_tc_tiling_on_sc=False),
        scratch_types=[
            pltpu.VMEM_SHARED((CROWS + CPAD, RANK), jnp.float32),
            pltpu.VMEM((CH,), jnp.int32),
            pltpu.VMEM((CH,), jnp.int32),
            pltpu.VMEM((CH,), jnp.int32),
            pltpu.VMEM((CH,), jnp.int32),
            pltpu.VMEM((CH,), jnp.int32),
            pltpu.VMEM((CH, RANK), jnp.float32),
            pltpu.SemaphoreType.DMA,
        ],
    )
    return sx_kernel, c_kernel


BLK = 400


def _pre_body(x_ref, b_ref, o_ref):
    o_ref[...] = jnp.dot(x_ref[...], b_ref[...],
                         preferred_element_type=jnp.float32)


def _post_body(x_ref, sx_ref, c2_ref, wt_ref, at_ref, w1t_ref, b1_ref,
               w2t_ref, b2_ref, w3t_ref, b3_ref, eps_ref, o_ref):
    f32 = jnp.float32
    sx = sx_ref[0] + sx_ref[1]
    out0 = (1.0 + eps_ref[0, 0]) * x_ref[...]
    out0 += jnp.dot(sx, wt_ref[...], preferred_element_type=f32)
    out0 += jnp.dot(c2_ref[...], at_ref[...], preferred_element_type=f32)
    h = jnp.maximum(jnp.dot(out0, w1t_ref[...], preferred_element_type=f32)
                    + b1_ref[...], 0.0)
    h = jnp.maximum(jnp.dot(h, w2t_ref[...], preferred_element_type=f32)
                    + b2_ref[...], 0.0)
    o_ref[...] = jnp.dot(h, w3t_ref[...], preferred_element_type=f32) \
        + b3_ref[...]


def _const_spec(shape):
    nd = len(shape)
    return pl.BlockSpec(shape, lambda i: (0,) * nd)


_pre_call = pl.pallas_call(
    _pre_body,
    grid=(N // BLK,),
    in_specs=[
        pl.BlockSpec((BLK, D), lambda i: (i, 0)),
        _const_spec((D, NUM_REL * RANK)),
    ],
    out_specs=pl.BlockSpec((BLK, NUM_REL * RANK), lambda i: (i, 0)),
    out_shape=jax.ShapeDtypeStruct((N, NUM_REL * RANK), jnp.float32),
)

_post_call = pl.pallas_call(
    _post_body,
    grid=(N // BLK,),
    in_specs=[
        pl.BlockSpec((BLK, D), lambda i: (i, 0)),
        pl.BlockSpec((NC, BLK, D), lambda i: (0, i, 0)),
        pl.BlockSpec((BLK, NUM_REL * RANK), lambda i: (i, 0)),
        _const_spec((D, D)),
        _const_spec((NUM_REL * RANK, D)),
        _const_spec((D, 2 * D)),
        _const_spec((1, 2 * D)),
        _const_spec((2 * D, 2 * D)),
        _const_spec((1, 2 * D)),
        _const_spec((2 * D, D)),
        _const_spec((1, D)),
        _const_spec((1, 1)),
    ],
    out_specs=pl.BlockSpec((BLK, D), lambda i: (i, 0)),
    out_shape=jax.ShapeDtypeStruct((N, D), jnp.float32),
)


@jax.jit
def kernel(x, edge_index, edge_type, W, eps, A_table, B_table,
           W1, b1, W2, b2, W3, b3):
    src = edge_index[0].astype(jnp.int32)
    dst = edge_index[1].astype(jnp.int32)
    rel = edge_type.astype(jnp.int32)

    b_full = B_table.reshape(NUM_REL, D, RANK).transpose(1, 0, 2) \
        .reshape(D, NUM_REL * RANK)
    a_full = A_table.reshape(NUM_REL, D, RANK).transpose(0, 2, 1) \
        .reshape(NUM_REL * RANK, D)

    sx_kernel, c_kernel = _sc_kernels()

    p = _pre_call(x, b_full)
    p_rows = p.reshape(N * NUM_REL, RANK)

    zeros_sx = jnp.zeros((N, D), jnp.float32)
    sx = sx_kernel(x, src, dst, zeros_sx)

    zeros_c = jnp.zeros((ZROWS, RANK), jnp.float32)
    cparts = c_kernel(p_rows, src, dst, rel, zeros_c)
    c2 = cparts.reshape(N, NUM_REL * RANK)

    return _post_call(x, sx, c2, W.T, a_full, W1.T, b1.reshape(1, -1),
                      W2.T, b2.reshape(1, -1), W3.T, b3.reshape(1, -1),
                      eps.reshape(1, 1))


# speedup vs baseline: 7.6357x; 1.2837x over previous
import functools

import jax
import jax.numpy as jnp
from jax import lax
from jax.experimental import pallas as pl
from jax.experimental.pallas import tpu as pltpu
from jax.experimental.pallas import tpu_sc as plsc

N = 10000
E = 320000
D = 128
RANK = 8
NUM_REL = 64

NC = 2
NS = 16
LANES = 16

CH = 128
NCHUNK = E // CH
NW = NC * NS

NSHARD = 4
SHN = N // NSHARD
CROWS = SHN * NUM_REL
CPAD = 16
ZROWS = CROWS // NS

ESPAN = E // NW
SXB = 2048
SXR = 1792
SXT = 16


def _sx_body(x_hbm, src_hbm, dst_hbm, zero_hbm, out_hbm,
             acc, sstage, dstage, didx, dtail, rows, rtail, sem):
    c = lax.axis_index("c")
    s = lax.axis_index("s")
    w = c * NS + s
    base = w * ESPAN

    ZB = 624
    pltpu.sync_copy(zero_hbm.at[pl.ds(s * ZB, ZB)], acc.at[pl.ds(s * ZB, ZB)])

    @pl.when(s == 0)
    def _():
        pltpu.sync_copy(zero_hbm.at[pl.ds(NS * ZB, N - NS * ZB)],
                        acc.at[pl.ds(NS * ZB, N - NS * ZB)])

    plsc.subcore_barrier()

    for b in range(ESPAN // SXB + 1):
        nch_b = SXB // CH if b < ESPAN // SXB else SXR // CH
        ne = nch_b * CH
        off = base + b * SXB
        pltpu.sync_copy(src_hbm.at[pl.ds(off, ne)], sstage.at[pl.ds(0, ne)])
        pltpu.sync_copy(dst_hbm.at[pl.ds(off, ne)], dstage.at[pl.ds(0, ne)])
        pltpu.async_copy(x_hbm.at[sstage.at[pl.ds(0, CH)]], rows.at[0], sem)

        @pl.loop(0, nch_b)
        def _(k):
            half = lax.rem(k, jnp.int32(2))
            pltpu.make_async_copy(
                x_hbm.at[sstage.at[pl.ds(k * CH, CH)]], rows.at[half],
                sem).wait()
            nxt = jnp.minimum(k + 1, nch_b - 1)
            pltpu.async_copy(x_hbm.at[sstage.at[pl.ds(nxt * CH, CH)]],
                             rows.at[1 - half], sem)
            for t in range(CH // LANES):
                didx[pl.ds(t * LANES, LANES)] = \
                    dstage[pl.ds(k * CH + t * LANES, LANES)]
            pltpu.sync_copy(rows.at[half], acc.at[didx], add=True)

        pltpu.make_async_copy(x_hbm.at[sstage.at[pl.ds(0, CH)]],
                              rows.at[0], sem).wait()

    toff = base + ESPAN - SXT
    pltpu.sync_copy(src_hbm.at[pl.ds(toff, SXT)], dtail)
    pltpu.async_copy(x_hbm.at[dtail], rtail, sem).wait()
    pltpu.sync_copy(dst_hbm.at[pl.ds(toff, SXT)], dtail)
    pltpu.sync_copy(rtail, acc.at[dtail], add=True)

    plsc.subcore_barrier()
    pltpu.sync_copy(acc.at[pl.ds(s * ZB, ZB)],
                    out_hbm.at[c].at[pl.ds(s * ZB, ZB)])

    @pl.when(s == 0)
    def _():
        pltpu.sync_copy(acc.at[pl.ds(NS * ZB, N - NS * ZB)],
                        out_hbm.at[c].at[pl.ds(NS * ZB, N - NS * ZB)])


CCH = 158
CSPAN = CCH * CH
E2 = CSPAN * NS
NGRP = CSPAN // LANES


def _c_body(p_hbm, gidx_hbm, cidx_hbm, zero_hbm, out_hbm,
            cacc, gidxb, cidxb, didx, prow, sem):
    c = lax.axis_index("c")
    s = lax.axis_index("s")
    base = s * CSPAN

    pltpu.sync_copy(gidx_hbm.at[pl.ds(base, CSPAN)], gidxb)

    for p in range(2):
        sh = c * 2 + p

        pltpu.sync_copy(zero_hbm, cacc.at[pl.ds(s * ZROWS, ZROWS)])

        @pl.when(s == 0)
        def _():
            pltpu.sync_copy(zero_hbm.at[pl.ds(0, CPAD)],
                            cacc.at[pl.ds(CROWS, CPAD)])

        pltpu.sync_copy(cidx_hbm.at[sh].at[pl.ds(base, CSPAN)], cidxb)

        pltpu.async_copy(p_hbm.at[gidxb.at[pl.ds(0, CH)]], prow.at[0], sem)
        plsc.subcore_barrier()

        @pl.loop(0, CCH)
        def _(k):
            half = lax.rem(k, jnp.int32(2))
            pltpu.make_async_copy(p_hbm.at[gidxb.at[pl.ds(k * CH, CH)]],
                                  prow.at[half], sem).wait()
            nxt = jnp.minimum(k + 1, CCH - 1)
            pltpu.async_copy(p_hbm.at[gidxb.at[pl.ds(nxt * CH, CH)]],
                             prow.at[1 - half], sem)
            for t in range(CH // LANES):
                didx[pl.ds(t * LANES, LANES)] = \
                    cidxb[pl.ds(k * CH + t * LANES, LANES)]
            pltpu.sync_copy(prow.at[half], cacc.at[didx], add=True)

        pltpu.make_async_copy(p_hbm.at[gidxb.at[pl.ds(0, CH)]],
                              prow.at[0], sem).wait()

        plsc.subcore_barrier()
        dump_rows = CROWS // NS
        pltpu.sync_copy(cacc.at[pl.ds(s * dump_rows, dump_rows)],
                        out_hbm.at[sh].at[pl.ds(s * dump_rows, dump_rows)])
        plsc.subcore_barrier()


@functools.lru_cache(maxsize=1)
def _sc_kernels():
    vmesh = plsc.VectorSubcoreMesh(core_axis_name="c", subcore_axis_name="s",
                                   num_cores=NC, num_subcores=NS)
    sx_kernel = pl.kernel(
        _sx_body,
        out_type=jax.ShapeDtypeStruct((NC, N, D), jnp.float32),
        mesh=vmesh,
        scratch_types=[
            pltpu.VMEM_SHARED((N, D), jnp.float32),
            pltpu.VMEM((SXB,), jnp.int32),
            pltpu.VMEM((SXB,), jnp.int32),
            pltpu.VMEM((CH,), jnp.int32),
            pltpu.VMEM((SXT,), jnp.int32),
            pltpu.VMEM((2, CH, D), jnp.float32),
            pltpu.VMEM((SXT, D), jnp.float32),
            pltpu.SemaphoreType.DMA,
        ],
    )
    c_kernel = pl.kernel(
        _c_body,
        out_type=jax.ShapeDtypeStruct((NSHARD, CROWS, RANK), jnp.float32),
        mesh=vmesh,
        compiler_params=pltpu.CompilerParams(use_tc_tiling_on_sc=False),
        scratch_types=[
            pltpu.VMEM_SHARED((CROWS + CPAD, RANK), jnp.float32),
            pltpu.VMEM((CSPAN,), jnp.int32),
            pltpu.VMEM((CSPAN,), jnp.int32),
            pltpu.VMEM((CH,), jnp.int32),
            pltpu.VMEM((2, CH, RANK), jnp.float32),
            pltpu.SemaphoreType.DMA,
        ],
    )
    return sx_kernel, c_kernel


BLK = 400


def _pre_body(x_ref, b_ref, o_ref):
    o_ref[...] = jnp.dot(x_ref[...], b_ref[...],
                         preferred_element_type=jnp.float32)


def _post_body(x_ref, sx_ref, c2_ref, wt_ref, at_ref, w1t_ref, b1_ref,
               w2t_ref, b2_ref, w3t_ref, b3_ref, eps_ref, o_ref):
    f32 = jnp.float32
    sx = sx_ref[0] + sx_ref[1]
    out0 = (1.0 + eps_ref[0, 0]) * x_ref[...]
    out0 += jnp.dot(sx, wt_ref[...], preferred_element_type=f32)
    out0 += jnp.dot(c2_ref[...], at_ref[...], preferred_element_type=f32)
    h = jnp.maximum(jnp.dot(out0, w1t_ref[...], preferred_element_type=f32)
                    + b1_ref[...], 0.0)
    h = jnp.maximum(jnp.dot(h, w2t_ref[...], preferred_element_type=f32)
                    + b2_ref[...], 0.0)
    o_ref[...] = jnp.dot(h, w3t_ref[...], preferred_element_type=f32) \
        + b3_ref[...]


def _const_spec(shape):
    nd = len(shape)
    return pl.BlockSpec(shape, lambda i: (0,) * nd)


_pre_call = pl.pallas_call(
    _pre_body,
    grid=(N // BLK,),
    in_specs=[
        pl.BlockSpec((BLK, D), lambda i: (i, 0)),
        _const_spec((D, NUM_REL * RANK)),
    ],
    out_specs=pl.BlockSpec((BLK, NUM_REL * RANK), lambda i: (i, 0)),
    out_shape=jax.ShapeDtypeStruct((N, NUM_REL * RANK), jnp.float32),
)

_post_call = pl.pallas_call(
    _post_body,
    grid=(N // BLK,),
    in_specs=[
        pl.BlockSpec((BLK, D), lambda i: (i, 0)),
        pl.BlockSpec((NC, BLK, D), lambda i: (0, i, 0)),
        pl.BlockSpec((BLK, NUM_REL * RANK), lambda i: (i, 0)),
        _const_spec((D, D)),
        _const_spec((NUM_REL * RANK, D)),
        _const_spec((D, 2 * D)),
        _const_spec((1, 2 * D)),
        _const_spec((2 * D, 2 * D)),
        _const_spec((1, 2 * D)),
        _const_spec((2 * D, D)),
        _const_spec((1, D)),
        _const_spec((1, 1)),
    ],
    out_specs=pl.BlockSpec((BLK, D), lambda i: (i, 0)),
    out_shape=jax.ShapeDtypeStruct((N, D), jnp.float32),
)


@jax.jit
def kernel(x, edge_index, edge_type, W, eps, A_table, B_table,
           W1, b1, W2, b2, W3, b3):
    src = edge_index[0].astype(jnp.int32)
    dst = edge_index[1].astype(jnp.int32)
    rel = edge_type.astype(jnp.int32)

    b_full = B_table.reshape(NUM_REL, D, RANK).transpose(1, 0, 2) \
        .reshape(D, NUM_REL * RANK)
    a_full = A_table.reshape(NUM_REL, D, RANK).transpose(0, 2, 1) \
        .reshape(NUM_REL * RANK, D)

    sx_kernel, c_kernel = _sc_kernels()

    p = _pre_call(x, b_full)
    p_rows = p.reshape(N * NUM_REL, RANK)

    zeros_sx = jnp.zeros((N, D), jnp.float32)
    sx = sx_kernel(x, src, dst, zeros_sx)

    pad = E2 - E
    src_p = jnp.concatenate([src, jnp.zeros((pad,), jnp.int32)])
    dst_p = jnp.concatenate([dst, jnp.full((pad,), N, jnp.int32)])
    rel_p = jnp.concatenate([rel, jnp.zeros((pad,), jnp.int32)])
    gidx = src_p * NUM_REL + rel_p
    shard_base = jnp.arange(NSHARD, dtype=jnp.int32)[:, None] * SHN
    loc = dst_p[None, :] - shard_base
    ok = (loc >= 0) & (loc < SHN)
    cidx = jnp.where(ok, loc * NUM_REL + rel_p[None, :], CROWS)

    zeros_c = jnp.zeros((ZROWS, RANK), jnp.float32)
    cparts = c_kernel(p_rows, gidx, cidx, zeros_c)
    c2 = cparts.reshape(N, NUM_REL * RANK)

    return _post_call(x, sx, c2, W.T, a_full, W1.T, b1.reshape(1, -1),
                      W2.T, b2.reshape(1, -1), W3.T, b3.reshape(1, -1),
                      eps.reshape(1, 1))


# speedup vs baseline: 7.9010x; 1.0347x over previous
import functools

import jax
import jax.numpy as jnp
from jax import lax
from jax.experimental import pallas as pl
from jax.experimental.pallas import tpu as pltpu
from jax.experimental.pallas import tpu_sc as plsc

N = 10000
E = 320000
D = 128
RANK = 8
NUM_REL = 64

NC = 2
NS = 16
LANES = 16

CH = 128
NCHUNK = E // CH
NW = NC * NS

NSHARD = 4
SHN = N // NSHARD
CROWS = SHN * NUM_REL
CPAD = 16
ZROWS = CROWS // NS

ESPAN = E // NW
SXB = 2048
SXR = 1792
SXT = 16


def _sx_body(x_hbm, src_hbm, dst_hbm, zero_hbm, out_hbm,
             acc, sstage, dstage, didx, dtail, rows, rtail, sem, sem2):
    c = lax.axis_index("c")
    s = lax.axis_index("s")
    w = c * NS + s
    base = w * ESPAN

    ZB = 624
    pltpu.sync_copy(zero_hbm.at[pl.ds(s * ZB, ZB)], acc.at[pl.ds(s * ZB, ZB)])

    @pl.when(s == 0)
    def _():
        pltpu.sync_copy(zero_hbm.at[pl.ds(NS * ZB, N - NS * ZB)],
                        acc.at[pl.ds(NS * ZB, N - NS * ZB)])

    plsc.subcore_barrier()

    def g_desc(blk_k, buf):
        return pltpu.make_async_copy(
            x_hbm.at[sstage.at[pl.ds(blk_k * CH, CH)]], rows.at[buf], sem)

    class s_desc:
        def __init__(self, buf):
            self.buf = buf

        def start(self):
            pltpu.async_copy(rows.at[self.buf], acc.at[didx.at[self.buf]],
                             sem2, add=True)

        def wait(self):
            pltpu.make_async_copy(rows.at[self.buf],
                                  acc.at[didx.at[self.buf]], sem2).wait()

    for b in range(ESPAN // SXB + 1):
        nch_b = SXB // CH if b < ESPAN // SXB else SXR // CH
        ne = nch_b * CH
        off = base + b * SXB
        pltpu.sync_copy(src_hbm.at[pl.ds(off, ne)], sstage.at[pl.ds(0, ne)])
        pltpu.sync_copy(dst_hbm.at[pl.ds(off, ne)], dstage.at[pl.ds(0, ne)])
        g_desc(0, 0).start()
        for k in range(nch_b):
            half = k % 2
            if k >= 1:
                s_desc(1 - half).wait()
            if k + 1 < nch_b:
                g_desc(k + 1, 1 - half).start()
            for t in range(CH // LANES):
                didx[half, pl.ds(t * LANES, LANES)] = \
                    dstage[pl.ds(k * CH + t * LANES, LANES)]
            g_desc(k, half).wait()
            s_desc(half).start()
        s_desc((nch_b - 1) % 2).wait()

    toff = base + ESPAN - SXT
    pltpu.sync_copy(src_hbm.at[pl.ds(toff, SXT)], dtail)
    pltpu.async_copy(x_hbm.at[dtail], rtail, sem).wait()
    pltpu.sync_copy(dst_hbm.at[pl.ds(toff, SXT)], dtail)
    pltpu.sync_copy(rtail, acc.at[dtail], add=True)

    plsc.subcore_barrier()
    pltpu.sync_copy(acc.at[pl.ds(s * ZB, ZB)],
                    out_hbm.at[c].at[pl.ds(s * ZB, ZB)])

    @pl.when(s == 0)
    def _():
        pltpu.sync_copy(acc.at[pl.ds(NS * ZB, N - NS * ZB)],
                        out_hbm.at[c].at[pl.ds(NS * ZB, N - NS * ZB)])


CCH = 158
CSPAN = CCH * CH
E2 = CSPAN * NS
NGRP = CSPAN // LANES


CRING = 6


def _c_body(p_hbm, gidx_hbm, cidx_hbm, zero_hbm, out_hbm,
            cacc, gidxb, cidxb, prow, sem, sem2):
    c = lax.axis_index("c")
    s = lax.axis_index("s")

    pltpu.sync_copy(gidx_hbm.at[pl.ds(s * CCH, CCH)], gidxb)

    def g_desc(k, buf):
        return pltpu.make_async_copy(p_hbm.at[gidxb.at[k]],
                                     prow.at[buf], sem)

    class s_desc:
        def __init__(self, k, buf):
            self.k, self.buf = k, buf

        def start(self):
            pltpu.async_copy(prow.at[self.buf], cacc.at[cidxb.at[self.k]],
                             sem2, add=True)

        def wait(self):
            pltpu.make_async_copy(prow.at[self.buf],
                                  cacc.at[cidxb.at[self.k]], sem2).wait()

    for p in range(2):
        sh = c * 2 + p

        pltpu.sync_copy(zero_hbm, cacc.at[pl.ds(s * ZROWS, ZROWS)])

        @pl.when(s == 0)
        def _():
            pltpu.sync_copy(zero_hbm.at[pl.ds(0, CPAD)],
                            cacc.at[pl.ds(CROWS, CPAD)])

        pltpu.sync_copy(cidx_hbm.at[sh].at[pl.ds(s * CCH, CCH)], cidxb)

        for j in range(CRING - 1):
            g_desc(j, j).start()
        plsc.subcore_barrier()

        g_desc(0, 0).wait()
        s_desc(0, 0).start()
        g_desc(CRING - 1, CRING - 1).start()

        @pl.loop(1, CCH)
        def _(k):
            bk = lax.rem(k, jnp.int32(CRING))
            bp = lax.rem(k - 1, jnp.int32(CRING))
            g_desc(k, bk).wait()
            s_desc(k, bk).start()
            s_desc(k - 1, bp).wait()
            nxt = jnp.minimum(k + CRING - 1, CCH - 1)
            g_desc(nxt, bp).start()

        s_desc(CCH - 1, (CCH - 1) % CRING).wait()
        for j in range(CRING - 1):
            g_desc(CCH - 1, j).wait()

        plsc.subcore_barrier()
        dump_rows = CROWS // NS
        pltpu.sync_copy(cacc.at[pl.ds(s * dump_rows, dump_rows)],
                        out_hbm.at[sh].at[pl.ds(s * dump_rows, dump_rows)])
        plsc.subcore_barrier()


@functools.lru_cache(maxsize=1)
def _sc_kernels():
    vmesh = plsc.VectorSubcoreMesh(core_axis_name="c", subcore_axis_name="s",
                                   num_cores=NC, num_subcores=NS)
    sx_kernel = pl.kernel(
        _sx_body,
        out_type=jax.ShapeDtypeStruct((NC, N, D), jnp.float32),
        mesh=vmesh,
        scratch_types=[
            pltpu.VMEM_SHARED((N, D), jnp.float32),
            pltpu.VMEM((SXB,), jnp.int32),
            pltpu.VMEM((SXB,), jnp.int32),
            pltpu.VMEM((2, CH), jnp.int32),
            pltpu.VMEM((SXT,), jnp.int32),
            pltpu.VMEM((2, CH, D), jnp.float32),
            pltpu.VMEM((SXT, D), jnp.float32),
            pltpu.SemaphoreType.DMA,
            pltpu.SemaphoreType.DMA,
        ],
    )
    c_kernel = pl.kernel(
        _c_body,
        out_type=jax.ShapeDtypeStruct((NSHARD, CROWS, RANK), jnp.float32),
        mesh=vmesh,
        compiler_params=pltpu.CompilerParams(use_tc_tiling_on_sc=False),
        scratch_types=[
            pltpu.VMEM_SHARED((CROWS + CPAD, RANK), jnp.float32),
            pltpu.VMEM((CCH, CH), jnp.int32),
            pltpu.VMEM((CCH, CH), jnp.int32),
            pltpu.VMEM((CRING, CH, RANK), jnp.float32),
            pltpu.SemaphoreType.DMA,
            pltpu.SemaphoreType.DMA,
        ],
    )
    return sx_kernel, c_kernel


BLK = 400


def _pre_body(x_ref, b_ref, o_ref):
    o_ref[...] = jnp.dot(x_ref[...], b_ref[...],
                         preferred_element_type=jnp.float32)


def _post_body(x_ref, sx_ref, c2_ref, wt_ref, at_ref, w1t_ref, b1_ref,
               w2t_ref, b2_ref, w3t_ref, b3_ref, eps_ref, o_ref):
    f32 = jnp.float32
    sx = sx_ref[0] + sx_ref[1]
    out0 = (1.0 + eps_ref[0, 0]) * x_ref[...]
    out0 += jnp.dot(sx, wt_ref[...], preferred_element_type=f32)
    out0 += jnp.dot(c2_ref[...], at_ref[...], preferred_element_type=f32)
    h = jnp.maximum(jnp.dot(out0, w1t_ref[...], preferred_element_type=f32)
                    + b1_ref[...], 0.0)
    h = jnp.maximum(jnp.dot(h, w2t_ref[...], preferred_element_type=f32)
                    + b2_ref[...], 0.0)
    o_ref[...] = jnp.dot(h, w3t_ref[...], preferred_element_type=f32) \
        + b3_ref[...]


def _const_spec(shape):
    nd = len(shape)
    return pl.BlockSpec(shape, lambda i: (0,) * nd)


_pre_call = pl.pallas_call(
    _pre_body,
    grid=(N // BLK,),
    in_specs=[
        pl.BlockSpec((BLK, D), lambda i: (i, 0)),
        _const_spec((D, NUM_REL * RANK)),
    ],
    out_specs=pl.BlockSpec((BLK, NUM_REL * RANK), lambda i: (i, 0)),
    out_shape=jax.ShapeDtypeStruct((N, NUM_REL * RANK), jnp.float32),
)

_post_call = pl.pallas_call(
    _post_body,
    grid=(N // BLK,),
    in_specs=[
        pl.BlockSpec((BLK, D), lambda i: (i, 0)),
        pl.BlockSpec((NC, BLK, D), lambda i: (0, i, 0)),
        pl.BlockSpec((BLK, NUM_REL * RANK), lambda i: (i, 0)),
        _const_spec((D, D)),
        _const_spec((NUM_REL * RANK, D)),
        _const_spec((D, 2 * D)),
        _const_spec((1, 2 * D)),
        _const_spec((2 * D, 2 * D)),
        _const_spec((1, 2 * D)),
        _const_spec((2 * D, D)),
        _const_spec((1, D)),
        _const_spec((1, 1)),
    ],
    out_specs=pl.BlockSpec((BLK, D), lambda i: (i, 0)),
    out_shape=jax.ShapeDtypeStruct((N, D), jnp.float32),
)


@jax.jit
def kernel(x, edge_index, edge_type, W, eps, A_table, B_table,
           W1, b1, W2, b2, W3, b3):
    src = edge_index[0].astype(jnp.int32)
    dst = edge_index[1].astype(jnp.int32)
    rel = edge_type.astype(jnp.int32)

    b_full = B_table.reshape(NUM_REL, D, RANK).transpose(1, 0, 2) \
        .reshape(D, NUM_REL * RANK)
    a_full = A_table.reshape(NUM_REL, D, RANK).transpose(0, 2, 1) \
        .reshape(NUM_REL * RANK, D)

    sx_kernel, c_kernel = _sc_kernels()

    p = _pre_call(x, b_full)
    p_rows = p.reshape(N * NUM_REL, RANK)

    zeros_sx = jnp.zeros((N, D), jnp.float32)
    sx = sx_kernel(x, src, dst, zeros_sx)

    pad = E2 - E
    src_p = jnp.concatenate([src, jnp.zeros((pad,), jnp.int32)])
    dst_p = jnp.concatenate([dst, jnp.full((pad,), N, jnp.int32)])
    rel_p = jnp.concatenate([rel, jnp.zeros((pad,), jnp.int32)])
    gidx = (src_p * NUM_REL + rel_p).reshape(NS * CCH, CH)
    shard_base = jnp.arange(NSHARD, dtype=jnp.int32)[:, None] * SHN
    loc = dst_p[None, :] - shard_base
    ok = (loc >= 0) & (loc < SHN)
    cidx = jnp.where(ok, loc * NUM_REL + rel_p[None, :],
                     CROWS).reshape(NSHARD, NS * CCH, CH)

    zeros_c = jnp.zeros((ZROWS, RANK), jnp.float32)
    cparts = c_kernel(p_rows, gidx, cidx, zeros_c)
    c2 = cparts.reshape(N, NUM_REL * RANK)

    return _post_call(x, sx, c2, W.T, a_full, W1.T, b1.reshape(1, -1),
                      W2.T, b2.reshape(1, -1), W3.T, b3.reshape(1, -1),
                      eps.reshape(1, 1))


# speedup vs baseline: 15.1683x; 1.9198x over previous
import functools

import jax
import jax.numpy as jnp
from jax import lax
from jax.experimental import pallas as pl
from jax.experimental.pallas import tpu as pltpu
from jax.experimental.pallas import tpu_sc as plsc

N = 10000
E = 320000
D = 128
RANK = 8
NUM_REL = 64

NC = 2
NS = 16
LANES = 16

CH = 128
NCHUNK = E // CH
NW = NC * NS

NSHARD = 4
SHN = N // NSHARD
CROWS = SHN * NUM_REL
CPAD = 16
ZROWS = CROWS // NS

ESPAN = E // NW
SXB = 2048
SXR = 1792
SXT = 16


def _sx_body(x_hbm, src_hbm, dst_hbm, zero_hbm, out_hbm,
             acc, sstage, dstage, didx, dtail, rows, rtail, sem, sem2):
    c = lax.axis_index("c")
    s = lax.axis_index("s")
    w = c * NS + s
    base = w * ESPAN

    ZB = 624
    pltpu.sync_copy(zero_hbm.at[pl.ds(s * ZB, ZB)], acc.at[pl.ds(s * ZB, ZB)])

    @pl.when(s == 0)
    def _():
        pltpu.sync_copy(zero_hbm.at[pl.ds(NS * ZB, N - NS * ZB)],
                        acc.at[pl.ds(NS * ZB, N - NS * ZB)])

    plsc.subcore_barrier()

    def g_desc(blk_k, buf):
        return pltpu.make_async_copy(
            x_hbm.at[sstage.at[pl.ds(blk_k * CH, CH)]], rows.at[buf], sem)

    class s_desc:
        def __init__(self, buf):
            self.buf = buf

        def start(self):
            pltpu.async_copy(rows.at[self.buf], acc.at[didx.at[self.buf]],
                             sem2, add=True)

        def wait(self):
            pltpu.make_async_copy(rows.at[self.buf],
                                  acc.at[didx.at[self.buf]], sem2).wait()

    for b in range(ESPAN // SXB + 1):
        nch_b = SXB // CH if b < ESPAN // SXB else SXR // CH
        ne = nch_b * CH
        off = base + b * SXB
        pltpu.sync_copy(src_hbm.at[pl.ds(off, ne)], sstage.at[pl.ds(0, ne)])
        pltpu.sync_copy(dst_hbm.at[pl.ds(off, ne)], dstage.at[pl.ds(0, ne)])
        g_desc(0, 0).start()
        for k in range(nch_b):
            half = k % 2
            if k >= 1:
                s_desc(1 - half).wait()
            if k + 1 < nch_b:
                g_desc(k + 1, 1 - half).start()
            for t in range(CH // LANES):
                didx[half, pl.ds(t * LANES, LANES)] = \
                    dstage[pl.ds(k * CH + t * LANES, LANES)]
            g_desc(k, half).wait()
            s_desc(half).start()
        s_desc((nch_b - 1) % 2).wait()

    toff = base + ESPAN - SXT
    pltpu.sync_copy(src_hbm.at[pl.ds(toff, SXT)], dtail)
    pltpu.async_copy(x_hbm.at[dtail], rtail, sem).wait()
    pltpu.sync_copy(dst_hbm.at[pl.ds(toff, SXT)], dtail)
    pltpu.sync_copy(rtail, acc.at[dtail], add=True)

    plsc.subcore_barrier()
    pltpu.sync_copy(acc.at[pl.ds(s * ZB, ZB)],
                    out_hbm.at[c].at[pl.ds(s * ZB, ZB)])

    @pl.when(s == 0)
    def _():
        pltpu.sync_copy(acc.at[pl.ds(NS * ZB, N - NS * ZB)],
                        out_hbm.at[c].at[pl.ds(NS * ZB, N - NS * ZB)])


CCH = 158
CSPAN = CCH * CH
E2 = CSPAN * NS
NGRP = CSPAN // LANES


CRING = 6
CCH2 = CCH + 2
CDUMP = (CCH2 - 1) * CH


def _c_body(p_hbm, gidx_hbm, cidx_hbm, zero_hbm, out_hbm,
            cacc, gidxb, cidxb, prow, sem, sem2):
    c = lax.axis_index("c")
    s = lax.axis_index("s")

    def g_desc(k, buf):
        return pltpu.make_async_copy(p_hbm.at[gidxb.at[k]],
                                     prow.at[buf], sem)

    class s_desc:
        def __init__(self, k, buf):
            self.k, self.buf = k, buf

        def start(self):
            pltpu.async_copy(prow.at[self.buf], cacc.at[cidxb.at[self.k]],
                             sem2, add=True)

        def wait(self):
            pltpu.make_async_copy(prow.at[self.buf],
                                  cacc.at[cidxb.at[self.k]], sem2).wait()

    for p in range(2):
        sh = c * 2 + p

        pltpu.sync_copy(zero_hbm, cacc.at[pl.ds(s * ZROWS, ZROWS)])

        @pl.when(s == 0)
        def _():
            pltpu.sync_copy(zero_hbm.at[pl.ds(0, CPAD)],
                            cacc.at[pl.ds(CROWS, CPAD)])

        pltpu.sync_copy(gidx_hbm.at[pl.ds(s * CCH, CCH)],
                        gidxb.at[pl.ds(0, CCH)])
        pltpu.sync_copy(cidx_hbm.at[sh].at[pl.ds(s * CCH, CCH)],
                        cidxb.at[pl.ds(0, CCH)])

        @pl.loop(0, NGRP, init_carry=jnp.int32(0))
        def scan(g, cnt):
            rows = lax.broadcast(lax.div(g, jnp.int32(CH // LANES)),
                                 (LANES,))
            cols = lax.rem(g, jnp.int32(CH // LANES)) * LANES \
                + lax.iota(jnp.int32, LANES)
            gv = plsc.load_gather(gidxb, [rows, cols])
            cv = plsc.load_gather(cidxb, [rows, cols])
            m = cv != jnp.int32(CROWS)
            mi = m.astype(jnp.int32)
            pos = jnp.where(m, cnt + plsc.cumsum(mi) - 1,
                            jnp.int32(CDUMP) + lax.iota(jnp.int32, LANES))
            pr = lax.shift_right_logical(pos, 7)
            pc = lax.bitwise_and(pos, jnp.int32(CH - 1))
            plsc.store_scatter(gidxb, [pr, pc], gv)
            plsc.store_scatter(cidxb, [pr, pc], cv)
            return cnt + jnp.sum(mi)

        cnt = scan
        for t in range(CH // LANES):
            pos = cnt + t * LANES + lax.iota(jnp.int32, LANES)
            pr = lax.shift_right_logical(pos, 7)
            pc = lax.bitwise_and(pos, jnp.int32(CH - 1))
            plsc.store_scatter(gidxb, [pr, pc],
                               jnp.zeros((LANES,), jnp.int32))
            plsc.store_scatter(cidxb, [pr, pc],
                               jnp.full((LANES,), CROWS, jnp.int32))
        nch = lax.div(cnt + jnp.int32(CH), jnp.int32(CH))

        for j in range(CRING - 1):
            g_desc(j, j).start()
        plsc.subcore_barrier()

        g_desc(0, 0).wait()
        s_desc(0, 0).start()
        g_desc(CRING - 1, CRING - 1).start()

        @pl.loop(1, nch)
        def _(k):
            bk = lax.rem(k, jnp.int32(CRING))
            bp = lax.rem(k - 1, jnp.int32(CRING))
            g_desc(k, bk).wait()
            s_desc(k, bk).start()
            s_desc(k - 1, bp).wait()
            nxt = jnp.minimum(k + CRING - 1, nch - 1)
            g_desc(nxt, bp).start()

        s_desc(nch - 1, lax.rem(nch - 1, jnp.int32(CRING))).wait()
        for j in range(CRING - 1):
            g_desc(0, j).wait()

        plsc.subcore_barrier()
        dump_rows = CROWS // NS
        pltpu.sync_copy(cacc.at[pl.ds(s * dump_rows, dump_rows)],
                        out_hbm.at[sh].at[pl.ds(s * dump_rows, dump_rows)])
        plsc.subcore_barrier()


@functools.lru_cache(maxsize=1)
def _sc_kernels():
    vmesh = plsc.VectorSubcoreMesh(core_axis_name="c", subcore_axis_name="s",
                                   num_cores=NC, num_subcores=NS)
    sx_kernel = pl.kernel(
        _sx_body,
        out_type=jax.ShapeDtypeStruct((NC, N, D), jnp.float32),
        mesh=vmesh,
        scratch_types=[
            pltpu.VMEM_SHARED((N, D), jnp.float32),
            pltpu.VMEM((SXB,), jnp.int32),
            pltpu.VMEM((SXB,), jnp.int32),
            pltpu.VMEM((2, CH), jnp.int32),
            pltpu.VMEM((SXT,), jnp.int32),
            pltpu.VMEM((2, CH, D), jnp.float32),
            pltpu.VMEM((SXT, D), jnp.float32),
            pltpu.SemaphoreType.DMA,
            pltpu.SemaphoreType.DMA,
        ],
    )
    c_kernel = pl.kernel(
        _c_body,
        out_type=jax.ShapeDtypeStruct((NSHARD, CROWS, RANK), jnp.float32),
        mesh=vmesh,
        compiler_params=pltpu.CompilerParams(use_tc_tiling_on_sc=False,
                                             needs_layout_passes=False),
        scratch_types=[
            pltpu.VMEM_SHARED((CROWS + CPAD, RANK), jnp.float32),
            pltpu.VMEM((CCH2, CH), jnp.int32),
            pltpu.VMEM((CCH2, CH), jnp.int32),
            pltpu.VMEM((CRING, CH, RANK), jnp.float32),
            pltpu.SemaphoreType.DMA,
            pltpu.SemaphoreType.DMA,
        ],
    )
    return sx_kernel, c_kernel


BLK = 400


def _pre_body(x_ref, b_ref, o_ref):
    o_ref[...] = jnp.dot(x_ref[...], b_ref[...],
                         preferred_element_type=jnp.float32)


def _post_body(x_ref, sx_ref, c2_ref, wt_ref, at_ref, w1t_ref, b1_ref,
               w2t_ref, b2_ref, w3t_ref, b3_ref, eps_ref, o_ref):
    f32 = jnp.float32
    sx = sx_ref[0] + sx_ref[1]
    out0 = (1.0 + eps_ref[0, 0]) * x_ref[...]
    out0 += jnp.dot(sx, wt_ref[...], preferred_element_type=f32)
    out0 += jnp.dot(c2_ref[...], at_ref[...], preferred_element_type=f32)
    h = jnp.maximum(jnp.dot(out0, w1t_ref[...], preferred_element_type=f32)
                    + b1_ref[...], 0.0)
    h = jnp.maximum(jnp.dot(h, w2t_ref[...], preferred_element_type=f32)
                    + b2_ref[...], 0.0)
    o_ref[...] = jnp.dot(h, w3t_ref[...], preferred_element_type=f32) \
        + b3_ref[...]


def _const_spec(shape):
    nd = len(shape)
    return pl.BlockSpec(shape, lambda i: (0,) * nd)


_pre_call = pl.pallas_call(
    _pre_body,
    grid=(N // BLK,),
    in_specs=[
        pl.BlockSpec((BLK, D), lambda i: (i, 0)),
        _const_spec((D, NUM_REL * RANK)),
    ],
    out_specs=pl.BlockSpec((BLK, NUM_REL * RANK), lambda i: (i, 0)),
    out_shape=jax.ShapeDtypeStruct((N, NUM_REL * RANK), jnp.float32),
)

_post_call = pl.pallas_call(
    _post_body,
    grid=(N // BLK,),
    in_specs=[
        pl.BlockSpec((BLK, D), lambda i: (i, 0)),
        pl.BlockSpec((NC, BLK, D), lambda i: (0, i, 0)),
        pl.BlockSpec((BLK, NUM_REL * RANK), lambda i: (i, 0)),
        _const_spec((D, D)),
        _const_spec((NUM_REL * RANK, D)),
        _const_spec((D, 2 * D)),
        _const_spec((1, 2 * D)),
        _const_spec((2 * D, 2 * D)),
        _const_spec((1, 2 * D)),
        _const_spec((2 * D, D)),
        _const_spec((1, D)),
        _const_spec((1, 1)),
    ],
    out_specs=pl.BlockSpec((BLK, D), lambda i: (i, 0)),
    out_shape=jax.ShapeDtypeStruct((N, D), jnp.float32),
)


@jax.jit
def kernel(x, edge_index, edge_type, W, eps, A_table, B_table,
           W1, b1, W2, b2, W3, b3):
    src = edge_index[0].astype(jnp.int32)
    dst = edge_index[1].astype(jnp.int32)
    rel = edge_type.astype(jnp.int32)

    b_full = B_table.reshape(NUM_REL, D, RANK).transpose(1, 0, 2) \
        .reshape(D, NUM_REL * RANK)
    a_full = A_table.reshape(NUM_REL, D, RANK).transpose(0, 2, 1) \
        .reshape(NUM_REL * RANK, D)

    sx_kernel, c_kernel = _sc_kernels()

    p = _pre_call(x, b_full)
    p_rows = p.reshape(N * NUM_REL, RANK)

    zeros_sx = jnp.zeros((N, D), jnp.float32)
    sx = sx_kernel(x, src, dst, zeros_sx)

    pad = E2 - E
    src_p = jnp.concatenate([src, jnp.zeros((pad,), jnp.int32)])
    dst_p = jnp.concatenate([dst, jnp.full((pad,), N, jnp.int32)])
    rel_p = jnp.concatenate([rel, jnp.zeros((pad,), jnp.int32)])
    gidx = (src_p * NUM_REL + rel_p).reshape(NS * CCH, CH)
    shard_base = jnp.arange(NSHARD, dtype=jnp.int32)[:, None] * SHN
    loc = dst_p[None, :] - shard_base
    ok = (loc >= 0) & (loc < SHN)
    cidx = jnp.where(ok, loc * NUM_REL + rel_p[None, :],
                     CROWS).reshape(NSHARD, NS * CCH, CH)

    zeros_c = jnp.zeros((ZROWS, RANK), jnp.float32)
    cparts = c_kernel(p_rows, gidx, cidx, zeros_c)
    c2 = cparts.reshape(N, NUM_REL * RANK)

    return _post_call(x, sx, c2, W.T, a_full, W1.T, b1.reshape(1, -1),
                      W2.T, b2.reshape(1, -1), W3.T, b3.reshape(1, -1),
                      eps.reshape(1, 1))


# speedup vs baseline: 15.4780x; 1.0204x over previous
import functools

import jax
import jax.numpy as jnp
from jax import lax
from jax.experimental import pallas as pl
from jax.experimental.pallas import tpu as pltpu
from jax.experimental.pallas import tpu_sc as plsc

N = 10000
E = 320000
D = 128
RANK = 8
NUM_REL = 64

NC = 2
NS = 16
LANES = 16

CH = 128
NCHUNK = E // CH
NW = NC * NS

NSHARD = 4
SHN = N // NSHARD
CROWS = SHN * NUM_REL
CPAD = 16
ZROWS = CROWS // NS

ESPAN = E // NW
SXB = 2048
SXR = 1792
SXT = 16


def _sx_body(x_hbm, src_hbm, dst_hbm, zero_hbm, out_hbm,
             acc, sstage, dstage, didx, dtail, rows, rtail, sem, sem2):
    c = lax.axis_index("c")
    s = lax.axis_index("s")
    w = c * NS + s
    base = w * ESPAN

    ZB = 624
    pltpu.sync_copy(zero_hbm.at[pl.ds(s * ZB, ZB)], acc.at[pl.ds(s * ZB, ZB)])

    @pl.when(s == 0)
    def _():
        pltpu.sync_copy(zero_hbm.at[pl.ds(NS * ZB, N - NS * ZB)],
                        acc.at[pl.ds(NS * ZB, N - NS * ZB)])

    plsc.subcore_barrier()

    def g_desc(blk_k, buf):
        return pltpu.make_async_copy(
            x_hbm.at[sstage.at[pl.ds(blk_k * CH, CH)]], rows.at[buf], sem)

    class s_desc:
        def __init__(self, buf):
            self.buf = buf

        def start(self):
            pltpu.async_copy(rows.at[self.buf], acc.at[didx.at[self.buf]],
                             sem2, add=True)

        def wait(self):
            pltpu.make_async_copy(rows.at[self.buf],
                                  acc.at[didx.at[self.buf]], sem2).wait()

    for b in range(ESPAN // SXB + 1):
        nch_b = SXB // CH if b < ESPAN // SXB else SXR // CH
        ne = nch_b * CH
        off = base + b * SXB
        pltpu.sync_copy(src_hbm.at[pl.ds(off, ne)], sstage.at[pl.ds(0, ne)])
        pltpu.sync_copy(dst_hbm.at[pl.ds(off, ne)], dstage.at[pl.ds(0, ne)])
        g_desc(0, 0).start()
        for k in range(nch_b):
            half = k % 2
            if k >= 1:
                s_desc(1 - half).wait()
            if k + 1 < nch_b:
                g_desc(k + 1, 1 - half).start()
            for t in range(CH // LANES):
                didx[half, pl.ds(t * LANES, LANES)] = \
                    dstage[pl.ds(k * CH + t * LANES, LANES)]
            g_desc(k, half).wait()
            s_desc(half).start()
        s_desc((nch_b - 1) % 2).wait()

    toff = base + ESPAN - SXT
    pltpu.sync_copy(src_hbm.at[pl.ds(toff, SXT)], dtail)
    pltpu.async_copy(x_hbm.at[dtail], rtail, sem).wait()
    pltpu.sync_copy(dst_hbm.at[pl.ds(toff, SXT)], dtail)
    pltpu.sync_copy(rtail, acc.at[dtail], add=True)

    plsc.subcore_barrier()
    pltpu.sync_copy(acc.at[pl.ds(s * ZB, ZB)],
                    out_hbm.at[c].at[pl.ds(s * ZB, ZB)])

    @pl.when(s == 0)
    def _():
        pltpu.sync_copy(acc.at[pl.ds(NS * ZB, N - NS * ZB)],
                        out_hbm.at[c].at[pl.ds(NS * ZB, N - NS * ZB)])


CCH = 158
CSPAN = CCH * CH
E2 = CSPAN * NS
NGRP = CSPAN // LANES


CRING = 6
CCH2 = CCH + 2
CDUMP = (CCH2 - 1) * CH


def _c_body(p_hbm, gidx_hbm, cidx_hbm, zero_hbm, out_hbm,
            cacc, gidxb, cidxb, prow, sem, sem2):
    c = lax.axis_index("c")
    s = lax.axis_index("s")

    def g_desc(k, buf):
        return pltpu.make_async_copy(p_hbm.at[gidxb.at[k]],
                                     prow.at[buf], sem)

    class s_desc:
        def __init__(self, k, buf):
            self.k, self.buf = k, buf

        def start(self):
            pltpu.async_copy(prow.at[self.buf], cacc.at[cidxb.at[self.k]],
                             sem2, add=True)

        def wait(self):
            pltpu.make_async_copy(prow.at[self.buf],
                                  cacc.at[cidxb.at[self.k]], sem2).wait()

    for p in range(2):
        sh = c * 2 + p

        zdesc = pltpu.make_async_copy(
            zero_hbm, cacc.at[pl.ds(s * ZROWS, ZROWS)], sem2)
        zdesc.start()

        pltpu.sync_copy(gidx_hbm.at[pl.ds(s * CCH, CCH)],
                        gidxb.at[pl.ds(0, CCH)])
        pltpu.sync_copy(cidx_hbm.at[sh].at[pl.ds(s * CCH, CCH)],
                        cidxb.at[pl.ds(0, CCH)])

        def compact_group(g, cnt):
            rows = lax.broadcast(lax.div(g, jnp.int32(CH // LANES)),
                                 (LANES,))
            cols = lax.rem(g, jnp.int32(CH // LANES)) * LANES \
                + lax.iota(jnp.int32, LANES)
            gv = plsc.load_gather(gidxb, [rows, cols])
            cv = plsc.load_gather(cidxb, [rows, cols])
            m = cv != jnp.int32(CROWS)
            mi = m.astype(jnp.int32)
            pos = jnp.where(m, cnt + plsc.cumsum(mi) - 1,
                            jnp.int32(CDUMP) + lax.iota(jnp.int32, LANES))
            pr = lax.shift_right_logical(pos, 7)
            pc = lax.bitwise_and(pos, jnp.int32(CH - 1))
            plsc.store_scatter(gidxb, [pr, pc], gv)
            plsc.store_scatter(cidxb, [pr, pc], cv)
            return cnt + jnp.sum(mi)

        @pl.loop(0, NGRP // 2, init_carry=jnp.int32(0))
        def scan(h, cnt):
            return compact_group(h * 2 + 1, compact_group(h * 2, cnt))

        cnt = scan
        zdesc.wait()

        @pl.when(s == 0)
        def _():
            pltpu.sync_copy(zero_hbm.at[pl.ds(0, CPAD)],
                            cacc.at[pl.ds(CROWS, CPAD)])
        for t in range(CH // LANES):
            pos = cnt + t * LANES + lax.iota(jnp.int32, LANES)
            pr = lax.shift_right_logical(pos, 7)
            pc = lax.bitwise_and(pos, jnp.int32(CH - 1))
            plsc.store_scatter(gidxb, [pr, pc],
                               jnp.zeros((LANES,), jnp.int32))
            plsc.store_scatter(cidxb, [pr, pc],
                               jnp.full((LANES,), CROWS, jnp.int32))
        nch = lax.div(cnt + jnp.int32(CH), jnp.int32(CH))

        for j in range(CRING - 1):
            g_desc(j, j).start()
        plsc.subcore_barrier()

        g_desc(0, 0).wait()
        s_desc(0, 0).start()
        g_desc(CRING - 1, CRING - 1).start()

        @pl.loop(1, nch)
        def _(k):
            bk = lax.rem(k, jnp.int32(CRING))
            bp = lax.rem(k - 1, jnp.int32(CRING))
            g_desc(k, bk).wait()
            s_desc(k, bk).start()
            s_desc(k - 1, bp).wait()
            nxt = jnp.minimum(k + CRING - 1, nch - 1)
            g_desc(nxt, bp).start()

        s_desc(nch - 1, lax.rem(nch - 1, jnp.int32(CRING))).wait()
        for j in range(CRING - 1):
            g_desc(0, j).wait()

        plsc.subcore_barrier()
        dump_rows = CROWS // NS
        pltpu.sync_copy(cacc.at[pl.ds(s * dump_rows, dump_rows)],
                        out_hbm.at[sh].at[pl.ds(s * dump_rows, dump_rows)])
        plsc.subcore_barrier()


@functools.lru_cache(maxsize=1)
def _sc_kernels():
    vmesh = plsc.VectorSubcoreMesh(core_axis_name="c", subcore_axis_name="s",
                                   num_cores=NC, num_subcores=NS)
    sx_kernel = pl.kernel(
        _sx_body,
        out_type=jax.ShapeDtypeStruct((NC, N, D), jnp.float32),
        mesh=vmesh,
        scratch_types=[
            pltpu.VMEM_SHARED((N, D), jnp.float32),
            pltpu.VMEM((SXB,), jnp.int32),
            pltpu.VMEM((SXB,), jnp.int32),
            pltpu.VMEM((2, CH), jnp.int32),
            pltpu.VMEM((SXT,), jnp.int32),
            pltpu.VMEM((2, CH, D), jnp.float32),
            pltpu.VMEM((SXT, D), jnp.float32),
            pltpu.SemaphoreType.DMA,
            pltpu.SemaphoreType.DMA,
        ],
    )
    c_kernel = pl.kernel(
        _c_body,
        out_type=jax.ShapeDtypeStruct((NSHARD, CROWS, RANK), jnp.float32),
        mesh=vmesh,
        compiler_params=pltpu.CompilerParams(use_tc_tiling_on_sc=False,
                                             needs_layout_passes=False),
        scratch_types=[
            pltpu.VMEM_SHARED((CROWS + CPAD, RANK), jnp.float32),
            pltpu.VMEM((CCH2, CH), jnp.int32),
            pltpu.VMEM((CCH2, CH), jnp.int32),
            pltpu.VMEM((CRING, CH, RANK), jnp.float32),
            pltpu.SemaphoreType.DMA,
            pltpu.SemaphoreType.DMA,
        ],
    )
    return sx_kernel, c_kernel


BLK = 400


def _pre_body(x_ref, b_ref, o_ref):
    o_ref[...] = jnp.dot(x_ref[...], b_ref[...],
                         preferred_element_type=jnp.float32)


def _post_body(x_ref, sx_ref, c2_ref, wt_ref, at_ref, w1t_ref, b1_ref,
               w2t_ref, b2_ref, w3t_ref, b3_ref, eps_ref, o_ref):
    f32 = jnp.float32
    sx = sx_ref[0] + sx_ref[1]
    out0 = (1.0 + eps_ref[0, 0]) * x_ref[...]
    out0 += jnp.dot(sx, wt_ref[...], preferred_element_type=f32)
    out0 += jnp.dot(c2_ref[...], at_ref[...], preferred_element_type=f32)
    h = jnp.maximum(jnp.dot(out0, w1t_ref[...], preferred_element_type=f32)
                    + b1_ref[...], 0.0)
    h = jnp.maximum(jnp.dot(h, w2t_ref[...], preferred_element_type=f32)
                    + b2_ref[...], 0.0)
    o_ref[...] = jnp.dot(h, w3t_ref[...], preferred_element_type=f32) \
        + b3_ref[...]


def _const_spec(shape):
    nd = len(shape)
    return pl.BlockSpec(shape, lambda i: (0,) * nd)


_pre_call = pl.pallas_call(
    _pre_body,
    grid=(N // BLK,),
    in_specs=[
        pl.BlockSpec((BLK, D), lambda i: (i, 0)),
        _const_spec((D, NUM_REL * RANK)),
    ],
    out_specs=pl.BlockSpec((BLK, NUM_REL * RANK), lambda i: (i, 0)),
    out_shape=jax.ShapeDtypeStruct((N, NUM_REL * RANK), jnp.float32),
)

_post_call = pl.pallas_call(
    _post_body,
    grid=(N // BLK,),
    in_specs=[
        pl.BlockSpec((BLK, D), lambda i: (i, 0)),
        pl.BlockSpec((NC, BLK, D), lambda i: (0, i, 0)),
        pl.BlockSpec((BLK, NUM_REL * RANK), lambda i: (i, 0)),
        _const_spec((D, D)),
        _const_spec((NUM_REL * RANK, D)),
        _const_spec((D, 2 * D)),
        _const_spec((1, 2 * D)),
        _const_spec((2 * D, 2 * D)),
        _const_spec((1, 2 * D)),
        _const_spec((2 * D, D)),
        _const_spec((1, D)),
        _const_spec((1, 1)),
    ],
    out_specs=pl.BlockSpec((BLK, D), lambda i: (i, 0)),
    out_shape=jax.ShapeDtypeStruct((N, D), jnp.float32),
)


@jax.jit
def kernel(x, edge_index, edge_type, W, eps, A_table, B_table,
           W1, b1, W2, b2, W3, b3):
    src = edge_index[0].astype(jnp.int32)
    dst = edge_index[1].astype(jnp.int32)
    rel = edge_type.astype(jnp.int32)

    b_full = B_table.reshape(NUM_REL, D, RANK).transpose(1, 0, 2) \
        .reshape(D, NUM_REL * RANK)
    a_full = A_table.reshape(NUM_REL, D, RANK).transpose(0, 2, 1) \
        .reshape(NUM_REL * RANK, D)

    sx_kernel, c_kernel = _sc_kernels()

    p = _pre_call(x, b_full)
    p_rows = p.reshape(N * NUM_REL, RANK)

    zeros_sx = jnp.zeros((N, D), jnp.float32)
    sx = sx_kernel(x, src, dst, zeros_sx)

    pad = E2 - E
    src_p = jnp.concatenate([src, jnp.zeros((pad,), jnp.int32)])
    dst_p = jnp.concatenate([dst, jnp.full((pad,), N, jnp.int32)])
    rel_p = jnp.concatenate([rel, jnp.zeros((pad,), jnp.int32)])
    gidx = (src_p * NUM_REL + rel_p).reshape(NS * CCH, CH)
    shard_base = jnp.arange(NSHARD, dtype=jnp.int32)[:, None] * SHN
    loc = dst_p[None, :] - shard_base
    ok = (loc >= 0) & (loc < SHN)
    cidx = jnp.where(ok, loc * NUM_REL + rel_p[None, :],
                     CROWS).reshape(NSHARD, NS * CCH, CH)

    zeros_c = jnp.zeros((ZROWS, RANK), jnp.float32)
    cparts = c_kernel(p_rows, gidx, cidx, zeros_c)
    c2 = cparts.reshape(N, NUM_REL * RANK)

    return _post_call(x, sx, c2, W.T, a_full, W1.T, b1.reshape(1, -1),
                      W2.T, b2.reshape(1, -1), W3.T, b3.reshape(1, -1),
                      eps.reshape(1, 1))


# speedup vs baseline: 15.5852x; 1.0069x over previous
import functools

import jax
import jax.numpy as jnp
from jax import lax
from jax.experimental import pallas as pl
from jax.experimental.pallas import tpu as pltpu
from jax.experimental.pallas import tpu_sc as plsc

N = 10000
E = 320000
D = 128
RANK = 8
NUM_REL = 64

NC = 2
NS = 16
LANES = 16

CH = 128
NCHUNK = E // CH
NW = NC * NS

NSHARD = 4
SHN = N // NSHARD
CROWS = SHN * NUM_REL
CPAD = 16
ZROWS = CROWS // NS

ESPAN = E // NW
SXB = 2048
SXR = 1792
SXT = 16


def _sx_body(x_hbm, src_hbm, dst_hbm, zero_hbm, out_hbm,
             acc, sstage, dstage, didx, dtail, rows, rtail, sem, sem2):
    c = lax.axis_index("c")
    s = lax.axis_index("s")
    w = c * NS + s
    base = w * ESPAN

    ZB = 624
    zdesc = pltpu.make_async_copy(zero_hbm.at[pl.ds(s * ZB, ZB)],
                                  acc.at[pl.ds(s * ZB, ZB)], sem2)
    zdesc.start()

    def g_desc(blk_k, buf):
        return pltpu.make_async_copy(
            x_hbm.at[sstage.at[pl.ds(blk_k * CH, CH)]], rows.at[buf], sem)

    class s_desc:
        def __init__(self, buf):
            self.buf = buf

        def start(self):
            pltpu.async_copy(rows.at[self.buf], acc.at[didx.at[self.buf]],
                             sem2, add=True)

        def wait(self):
            pltpu.make_async_copy(rows.at[self.buf],
                                  acc.at[didx.at[self.buf]], sem2).wait()

    for b in range(ESPAN // SXB + 1):
        nch_b = SXB // CH if b < ESPAN // SXB else SXR // CH
        ne = nch_b * CH
        off = base + b * SXB
        pltpu.sync_copy(src_hbm.at[pl.ds(off, ne)], sstage.at[pl.ds(0, ne)])
        pltpu.sync_copy(dst_hbm.at[pl.ds(off, ne)], dstage.at[pl.ds(0, ne)])
        g_desc(0, 0).start()
        if b == 0:
            zdesc.wait()

            @pl.when(s == 0)
            def _():
                pltpu.sync_copy(zero_hbm.at[pl.ds(NS * ZB, N - NS * ZB)],
                                acc.at[pl.ds(NS * ZB, N - NS * ZB)])

            plsc.subcore_barrier()
        for k in range(nch_b):
            half = k % 2
            if k >= 1:
                s_desc(1 - half).wait()
            if k + 1 < nch_b:
                g_desc(k + 1, 1 - half).start()
            for t in range(CH // LANES):
                didx[half, pl.ds(t * LANES, LANES)] = \
                    dstage[pl.ds(k * CH + t * LANES, LANES)]
            g_desc(k, half).wait()
            s_desc(half).start()
        s_desc((nch_b - 1) % 2).wait()

    toff = base + ESPAN - SXT
    pltpu.sync_copy(src_hbm.at[pl.ds(toff, SXT)], dtail)
    pltpu.async_copy(x_hbm.at[dtail], rtail, sem).wait()
    pltpu.sync_copy(dst_hbm.at[pl.ds(toff, SXT)], dtail)
    pltpu.sync_copy(rtail, acc.at[dtail], add=True)

    plsc.subcore_barrier()
    pltpu.sync_copy(acc.at[pl.ds(s * ZB, ZB)],
                    out_hbm.at[c].at[pl.ds(s * ZB, ZB)])

    @pl.when(s == 0)
    def _():
        pltpu.sync_copy(acc.at[pl.ds(NS * ZB, N - NS * ZB)],
                        out_hbm.at[c].at[pl.ds(NS * ZB, N - NS * ZB)])


CCH = 158
CSPAN = CCH * CH
E2 = CSPAN * NS
NGRP = CSPAN // LANES


CRING = 6
CCH2 = CCH + 2
CDUMP = (CCH2 - 1) * CH


def _c_body(p_hbm, gidx_hbm, cidx_hbm, zero_hbm, out_hbm,
            cacc, gidxb, cidxb, prow, sem, sem2):
    c = lax.axis_index("c")
    s = lax.axis_index("s")

    def g_desc(k, buf):
        return pltpu.make_async_copy(p_hbm.at[gidxb.at[k]],
                                     prow.at[buf], sem)

    class s_desc:
        def __init__(self, k, buf):
            self.k, self.buf = k, buf

        def start(self):
            pltpu.async_copy(prow.at[self.buf], cacc.at[cidxb.at[self.k]],
                             sem2, add=True)

        def wait(self):
            pltpu.make_async_copy(prow.at[self.buf],
                                  cacc.at[cidxb.at[self.k]], sem2).wait()

    for p in range(2):
        sh = c * 2 + p

        zdesc = pltpu.make_async_copy(
            zero_hbm, cacc.at[pl.ds(s * ZROWS, ZROWS)], sem2)
        zdesc.start()

        pltpu.sync_copy(gidx_hbm.at[pl.ds(s * CCH, CCH)],
                        gidxb.at[pl.ds(0, CCH)])
        pltpu.sync_copy(cidx_hbm.at[sh].at[pl.ds(s * CCH, CCH)],
                        cidxb.at[pl.ds(0, CCH)])

        def compact_group(g, cnt):
            rows = lax.broadcast(lax.div(g, jnp.int32(CH // LANES)),
                                 (LANES,))
            cols = lax.rem(g, jnp.int32(CH // LANES)) * LANES \
                + lax.iota(jnp.int32, LANES)
            gv = plsc.load_gather(gidxb, [rows, cols])
            cv = plsc.load_gather(cidxb, [rows, cols])
            m = cv != jnp.int32(CROWS)
            mi = m.astype(jnp.int32)
            pos = jnp.where(m, cnt + plsc.cumsum(mi) - 1,
                            jnp.int32(CDUMP) + lax.iota(jnp.int32, LANES))
            pr = lax.shift_right_logical(pos, 7)
            pc = lax.bitwise_and(pos, jnp.int32(CH - 1))
            plsc.store_scatter(gidxb, [pr, pc], gv)
            plsc.store_scatter(cidxb, [pr, pc], cv)
            return cnt + jnp.sum(mi)

        @pl.loop(0, NGRP // 2, init_carry=jnp.int32(0))
        def scan(h, cnt):
            return compact_group(h * 2 + 1, compact_group(h * 2, cnt))

        cnt = scan
        zdesc.wait()

        @pl.when(s == 0)
        def _():
            pltpu.sync_copy(zero_hbm.at[pl.ds(0, CPAD)],
                            cacc.at[pl.ds(CROWS, CPAD)])
        for t in range(CH // LANES):
            pos = cnt + t * LANES + lax.iota(jnp.int32, LANES)
            pr = lax.shift_right_logical(pos, 7)
            pc = lax.bitwise_and(pos, jnp.int32(CH - 1))
            plsc.store_scatter(gidxb, [pr, pc],
                               jnp.zeros((LANES,), jnp.int32))
            plsc.store_scatter(cidxb, [pr, pc],
                               jnp.full((LANES,), CROWS, jnp.int32))
        nch = lax.div(cnt + jnp.int32(CH), jnp.int32(CH))

        for j in range(CRING - 1):
            g_desc(j, j).start()
        plsc.subcore_barrier()

        g_desc(0, 0).wait()
        s_desc(0, 0).start()
        g_desc(CRING - 1, CRING - 1).start()

        @pl.loop(1, nch)
        def _(k):
            bk = lax.rem(k, jnp.int32(CRING))
            bp = lax.rem(k - 1, jnp.int32(CRING))
            g_desc(k, bk).wait()
            s_desc(k, bk).start()
            s_desc(k - 1, bp).wait()
            nxt = jnp.minimum(k + CRING - 1, nch - 1)
            g_desc(nxt, bp).start()

        s_desc(nch - 1, lax.rem(nch - 1, jnp.int32(CRING))).wait()
        for j in range(CRING - 1):
            g_desc(0, j).wait()

        plsc.subcore_barrier()
        dump_rows = CROWS // NS
        pltpu.sync_copy(cacc.at[pl.ds(s * dump_rows, dump_rows)],
                        out_hbm.at[sh].at[pl.ds(s * dump_rows, dump_rows)])
        plsc.subcore_barrier()


@functools.lru_cache(maxsize=1)
def _sc_kernels():
    vmesh = plsc.VectorSubcoreMesh(core_axis_name="c", subcore_axis_name="s",
                                   num_cores=NC, num_subcores=NS)
    sx_kernel = pl.kernel(
        _sx_body,
        out_type=jax.ShapeDtypeStruct((NC, N, D), jnp.float32),
        mesh=vmesh,
        scratch_types=[
            pltpu.VMEM_SHARED((N, D), jnp.float32),
            pltpu.VMEM((SXB,), jnp.int32),
            pltpu.VMEM((SXB,), jnp.int32),
            pltpu.VMEM((2, CH), jnp.int32),
            pltpu.VMEM((SXT,), jnp.int32),
            pltpu.VMEM((2, CH, D), jnp.float32),
            pltpu.VMEM((SXT, D), jnp.float32),
            pltpu.SemaphoreType.DMA,
            pltpu.SemaphoreType.DMA,
        ],
    )
    c_kernel = pl.kernel(
        _c_body,
        out_type=jax.ShapeDtypeStruct((NSHARD, CROWS, RANK), jnp.float32),
        mesh=vmesh,
        compiler_params=pltpu.CompilerParams(use_tc_tiling_on_sc=False,
                                             needs_layout_passes=False),
        scratch_types=[
            pltpu.VMEM_SHARED((CROWS + CPAD, RANK), jnp.float32),
            pltpu.VMEM((CCH2, CH), jnp.int32),
            pltpu.VMEM((CCH2, CH), jnp.int32),
            pltpu.VMEM((CRING, CH, RANK), jnp.float32),
            pltpu.SemaphoreType.DMA,
            pltpu.SemaphoreType.DMA,
        ],
    )
    return sx_kernel, c_kernel


BLK = 400


def _pre_body(x_ref, b_ref, o_ref):
    o_ref[...] = jnp.dot(x_ref[...], b_ref[...],
                         preferred_element_type=jnp.float32)


def _post_body(x_ref, sx_ref, c2_ref, wt_ref, at_ref, w1t_ref, b1_ref,
               w2t_ref, b2_ref, w3t_ref, b3_ref, eps_ref, o_ref):
    f32 = jnp.float32
    sx = sx_ref[0] + sx_ref[1]
    out0 = (1.0 + eps_ref[0, 0]) * x_ref[...]
    out0 += jnp.dot(sx, wt_ref[...], preferred_element_type=f32)
    out0 += jnp.dot(c2_ref[...], at_ref[...], preferred_element_type=f32)
    h = jnp.maximum(jnp.dot(out0, w1t_ref[...], preferred_element_type=f32)
                    + b1_ref[...], 0.0)
    h = jnp.maximum(jnp.dot(h, w2t_ref[...], preferred_element_type=f32)
                    + b2_ref[...], 0.0)
    o_ref[...] = jnp.dot(h, w3t_ref[...], preferred_element_type=f32) \
        + b3_ref[...]


def _const_spec(shape):
    nd = len(shape)
    return pl.BlockSpec(shape, lambda i: (0,) * nd)


_pre_call = pl.pallas_call(
    _pre_body,
    grid=(N // BLK,),
    in_specs=[
        pl.BlockSpec((BLK, D), lambda i: (i, 0)),
        _const_spec((D, NUM_REL * RANK)),
    ],
    out_specs=pl.BlockSpec((BLK, NUM_REL * RANK), lambda i: (i, 0)),
    out_shape=jax.ShapeDtypeStruct((N, NUM_REL * RANK), jnp.float32),
)

_post_call = pl.pallas_call(
    _post_body,
    grid=(N // BLK,),
    in_specs=[
        pl.BlockSpec((BLK, D), lambda i: (i, 0)),
        pl.BlockSpec((NC, BLK, D), lambda i: (0, i, 0)),
        pl.BlockSpec((BLK, NUM_REL * RANK), lambda i: (i, 0)),
        _const_spec((D, D)),
        _const_spec((NUM_REL * RANK, D)),
        _const_spec((D, 2 * D)),
        _const_spec((1, 2 * D)),
        _const_spec((2 * D, 2 * D)),
        _const_spec((1, 2 * D)),
        _const_spec((2 * D, D)),
        _const_spec((1, D)),
        _const_spec((1, 1)),
    ],
    out_specs=pl.BlockSpec((BLK, D), lambda i: (i, 0)),
    out_shape=jax.ShapeDtypeStruct((N, D), jnp.float32),
)


@jax.jit
def kernel(x, edge_index, edge_type, W, eps, A_table, B_table,
           W1, b1, W2, b2, W3, b3):
    src = edge_index[0].astype(jnp.int32)
    dst = edge_index[1].astype(jnp.int32)
    rel = edge_type.astype(jnp.int32)

    b_full = B_table.reshape(NUM_REL, D, RANK).transpose(1, 0, 2) \
        .reshape(D, NUM_REL * RANK)
    a_full = A_table.reshape(NUM_REL, D, RANK).transpose(0, 2, 1) \
        .reshape(NUM_REL * RANK, D)

    sx_kernel, c_kernel = _sc_kernels()

    p = _pre_call(x, b_full)
    p_rows = p.reshape(N * NUM_REL, RANK)

    zeros_sx = jnp.zeros((N, D), jnp.float32)
    sx = sx_kernel(x, src, dst, zeros_sx)

    pad = E2 - E
    src_p = jnp.concatenate([src, jnp.zeros((pad,), jnp.int32)])
    dst_p = jnp.concatenate([dst, jnp.full((pad,), N, jnp.int32)])
    rel_p = jnp.concatenate([rel, jnp.zeros((pad,), jnp.int32)])
    gidx = (src_p * NUM_REL + rel_p).reshape(NS * CCH, CH)
    shard_base = jnp.arange(NSHARD, dtype=jnp.int32)[:, None] * SHN
    loc = dst_p[None, :] - shard_base
    ok = (loc >= 0) & (loc < SHN)
    cidx = jnp.where(ok, loc * NUM_REL + rel_p[None, :],
                     CROWS).reshape(NSHARD, NS * CCH, CH)

    zeros_c = jnp.zeros((ZROWS, RANK), jnp.float32)
    cparts = c_kernel(p_rows, gidx, cidx, zeros_c)
    c2 = cparts.reshape(N, NUM_REL * RANK)

    return _post_call(x, sx, c2, W.T, a_full, W1.T, b1.reshape(1, -1),
                      W2.T, b2.reshape(1, -1), W3.T, b3.reshape(1, -1),
                      eps.reshape(1, 1))
